# Initial kernel scaffold; baseline (speedup 1.0000x reference)
import jax
import jax.numpy as jnp
from jax.experimental import pallas as pl


def kernel(input):
    raise NotImplementedError("write your pallas kernel here")


# speedup vs baseline: 4.3070x; 4.3070x over previous
import functools

import jax
import jax.numpy as jnp
from jax import lax
from jax.experimental import pallas as pl
from jax.experimental.pallas import tpu as pltpu
from jax.experimental.pallas import tpu_sc as plsc

NUM_NEG = 20
CROSS_NEG = 5
TOTAL_NEG = NUM_NEG + CROSS_NEG
BSZ, SEQ, DIM = 2, 2048, 768
NUM_ROWS = BSZ * SEQ
NUM_OUT = TOTAL_NEG * BSZ * SEQ

NC, NS = 2, 16
NW = NC * NS
ROWS_PER_W = NUM_OUT // NW
CHUNK = 128
NCHUNKS = ROWS_PER_W // CHUNK


def _neg_indices():
    key = jax.random.key(42)
    k1, key = jax.random.split(key)
    tszs = jnp.repeat(jnp.arange(SEQ), NUM_NEG)
    neg = jax.random.randint(k1, (BSZ, NUM_NEG * SEQ), 0, SEQ - 1)
    neg = jnp.where(neg >= tszs[None, :], neg + 1, neg)
    neg = neg + jnp.arange(BSZ)[:, None] * SEQ
    k2, key = jax.random.split(key)
    tszs_c = jnp.repeat(jnp.arange(SEQ), CROSS_NEG)
    cross = jax.random.randint(k2, (BSZ, CROSS_NEG * SEQ), 0, BSZ * SEQ - 1)
    cross = jnp.where(cross >= tszs_c[None, :], cross + 1, cross)
    return jnp.concatenate([neg, cross], axis=1)


@functools.cache
def _build_gather_sc():
    @functools.partial(
        pl.kernel,
        out_type=jax.ShapeDtypeStruct((NUM_OUT, DIM), jnp.float32),
        mesh=plsc.VectorSubcoreMesh(core_axis_name="c", subcore_axis_name="s"),
        scratch_types=[
            pltpu.VMEM((ROWS_PER_W,), jnp.int32),
            pltpu.VMEM((CHUNK, DIM), jnp.float32),
            pltpu.SemaphoreType.DMA,
        ],
    )
    def _gather_sc(table_hbm, idx_hbm, out_hbm, idx_v, rows_v, sem):
        wid = lax.axis_index("s") * NC + lax.axis_index("c")
        base = pl.multiple_of(wid * ROWS_PER_W, ROWS_PER_W)
        pltpu.sync_copy(idx_hbm.at[pl.ds(base, ROWS_PER_W)], idx_v)

        def chunk_body(k, carry):
            off = pl.multiple_of(k * CHUNK, CHUNK)
            pltpu.async_copy(
                table_hbm.at[idx_v.at[pl.ds(off, CHUNK)]], rows_v, sem
            ).wait()
            pltpu.sync_copy(rows_v, out_hbm.at[pl.ds(base + off, CHUNK)])
            return carry

        lax.fori_loop(0, NCHUNKS, chunk_body, 0)

    return _gather_sc


def kernel(input):
    x = input
    neg_idxs = _neg_indices()
    perm = jnp.transpose(
        neg_idxs.reshape(BSZ, SEQ, TOTAL_NEG), (2, 0, 1)
    ).reshape(-1)
    flat = x.reshape(NUM_ROWS, DIM)
    out = _build_gather_sc()(flat, perm)
    negs = out.reshape(TOTAL_NEG, BSZ, SEQ, DIM)
    return (x, negs, neg_idxs)


# speedup vs baseline: 4.3902x; 1.0193x over previous
import functools

import jax
import jax.numpy as jnp
from jax import lax
from jax.experimental import pallas as pl
from jax.experimental.pallas import tpu as pltpu
from jax.experimental.pallas import tpu_sc as plsc

NUM_NEG = 20
CROSS_NEG = 5
TOTAL_NEG = NUM_NEG + CROSS_NEG
BSZ, SEQ, DIM = 2, 2048, 768
NUM_ROWS = BSZ * SEQ
NUM_OUT = TOTAL_NEG * BSZ * SEQ

NC, NS = 2, 16
NW = NC * NS
ROWS_PER_W = NUM_OUT // NW
CHUNK = 64
NCHUNKS = ROWS_PER_W // CHUNK


def _neg_indices():
    key = jax.random.key(42)
    k1, key = jax.random.split(key)
    tszs = jnp.repeat(jnp.arange(SEQ), NUM_NEG)
    neg = jax.random.randint(k1, (BSZ, NUM_NEG * SEQ), 0, SEQ - 1)
    neg = jnp.where(neg >= tszs[None, :], neg + 1, neg)
    neg = neg + jnp.arange(BSZ)[:, None] * SEQ
    k2, key = jax.random.split(key)
    tszs_c = jnp.repeat(jnp.arange(SEQ), CROSS_NEG)
    cross = jax.random.randint(k2, (BSZ, CROSS_NEG * SEQ), 0, BSZ * SEQ - 1)
    cross = jnp.where(cross >= tszs_c[None, :], cross + 1, cross)
    return jnp.concatenate([neg, cross], axis=1)


@functools.cache
def _build_gather_sc():
    @functools.partial(
        pl.kernel,
        out_type=jax.ShapeDtypeStruct((NUM_OUT, DIM), jnp.float32),
        mesh=plsc.VectorSubcoreMesh(core_axis_name="c", subcore_axis_name="s"),
        scratch_types=[
            pltpu.VMEM((ROWS_PER_W,), jnp.int32),
            pltpu.VMEM((CHUNK, DIM), jnp.float32),
            pltpu.VMEM((CHUNK, DIM), jnp.float32),
            pltpu.SemaphoreType.DMA,
            pltpu.SemaphoreType.DMA,
            pltpu.SemaphoreType.DMA,
            pltpu.SemaphoreType.DMA,
        ],
    )
    def _gather_sc(table_hbm, idx_hbm, out_hbm, idx_v, rows0, rows1,
                   sg0, sg1, sw0, sw1):
        wid = lax.axis_index("s") * NC + lax.axis_index("c")
        base = pl.multiple_of(wid * ROWS_PER_W, ROWS_PER_W)
        pltpu.sync_copy(idx_hbm.at[pl.ds(base, ROWS_PER_W)], idx_v)

        bufs = (rows0, rows1)
        sgs = (sg0, sg1)
        sws = (sw0, sw1)

        def gather_start(k, b):
            off = pl.multiple_of(k * CHUNK, CHUNK)
            pltpu.async_copy(
                table_hbm.at[idx_v.at[pl.ds(off, CHUNK)]], bufs[b], sgs[b]
            )

        def gather_wait(b):
            pltpu.make_async_copy(
                table_hbm.at[idx_v.at[pl.ds(0, CHUNK)]], bufs[b], sgs[b]
            ).wait()

        def write_start(k, b):
            off = pl.multiple_of(k * CHUNK, CHUNK)
            pltpu.async_copy(
                bufs[b], out_hbm.at[pl.ds(base + off, CHUNK)], sws[b]
            )

        def write_wait(b):
            pltpu.make_async_copy(
                bufs[b], out_hbm.at[pl.ds(base, CHUNK)], sws[b]
            ).wait()

        gather_start(0, 0)
        gather_wait(0)
        write_start(0, 0)
        gather_start(1, 1)

        def group(jj, carry):
            for i in range(2):
                k = 1 + 2 * jj + i
                b = (1 + i) % 2
                gather_wait(b)
                write_start(k, b)
                write_wait(1 - b)
                gather_start(k + 1, 1 - b)
            return carry

        lax.fori_loop(0, (NCHUNKS - 2) // 2, group, 0)

        gather_wait(1)
        write_start(NCHUNKS - 1, 1)
        write_wait(0)
        write_wait(1)

    return _gather_sc


def kernel(input):
    x = input
    neg_idxs = _neg_indices()
    perm = jnp.transpose(
        neg_idxs.reshape(BSZ, SEQ, TOTAL_NEG), (2, 0, 1)
    ).reshape(-1)
    flat = x.reshape(NUM_ROWS, DIM)
    out = _build_gather_sc()(flat, perm)
    negs = out.reshape(TOTAL_NEG, BSZ, SEQ, DIM)
    return (x, negs, neg_idxs)


# speedup vs baseline: 4.3911x; 1.0002x over previous
import functools

import jax
import jax.numpy as jnp
from jax import lax
from jax.experimental import pallas as pl
from jax.experimental.pallas import tpu as pltpu
from jax.experimental.pallas import tpu_sc as plsc

NUM_NEG = 20
CROSS_NEG = 5
TOTAL_NEG = NUM_NEG + CROSS_NEG
BSZ, SEQ, DIM = 2, 2048, 768
NUM_ROWS = BSZ * SEQ
NUM_OUT = TOTAL_NEG * BSZ * SEQ

NC, NS = 2, 16
NW = NC * NS
ROWS_PER_W = NUM_OUT // NW
CHUNK = 80
NCHUNKS = ROWS_PER_W // CHUNK


def _neg_indices():
    key = jax.random.key(42)
    k1, key = jax.random.split(key)
    tszs = jnp.repeat(jnp.arange(SEQ), NUM_NEG)
    neg = jax.random.randint(k1, (BSZ, NUM_NEG * SEQ), 0, SEQ - 1)
    neg = jnp.where(neg >= tszs[None, :], neg + 1, neg)
    neg = neg + jnp.arange(BSZ)[:, None] * SEQ
    k2, key = jax.random.split(key)
    tszs_c = jnp.repeat(jnp.arange(SEQ), CROSS_NEG)
    cross = jax.random.randint(k2, (BSZ, CROSS_NEG * SEQ), 0, BSZ * SEQ - 1)
    cross = jnp.where(cross >= tszs_c[None, :], cross + 1, cross)
    return jnp.concatenate([neg, cross], axis=1)


@functools.cache
def _build_gather_sc():
    @functools.partial(
        pl.kernel,
        out_type=jax.ShapeDtypeStruct((NUM_OUT, DIM), jnp.float32),
        mesh=plsc.VectorSubcoreMesh(core_axis_name="c", subcore_axis_name="s"),
        scratch_types=[
            pltpu.VMEM((ROWS_PER_W,), jnp.int32),
            pltpu.VMEM((CHUNK, DIM), jnp.float32),
            pltpu.VMEM((CHUNK, DIM), jnp.float32),
            pltpu.SemaphoreType.DMA,
            pltpu.SemaphoreType.DMA,
            pltpu.SemaphoreType.DMA,
            pltpu.SemaphoreType.DMA,
        ],
    )
    def _gather_sc(table_hbm, idx_hbm, out_hbm, idx_v, rows0, rows1,
                   sg0, sg1, sw0, sw1):
        wid = lax.axis_index("s") * NC + lax.axis_index("c")
        base = pl.multiple_of(wid * ROWS_PER_W, ROWS_PER_W)
        pltpu.sync_copy(idx_hbm.at[pl.ds(base, ROWS_PER_W)], idx_v)

        bufs = (rows0, rows1)
        sgs = (sg0, sg1)
        sws = (sw0, sw1)

        def gather_start(k, b):
            off = pl.multiple_of(k * CHUNK, CHUNK)
            pltpu.async_copy(
                table_hbm.at[idx_v.at[pl.ds(off, CHUNK)]], bufs[b], sgs[b]
            )

        def gather_wait(b):
            pltpu.make_async_copy(
                table_hbm.at[idx_v.at[pl.ds(0, CHUNK)]], bufs[b], sgs[b]
            ).wait()

        def write_start(k, b):
            off = pl.multiple_of(k * CHUNK, CHUNK)
            pltpu.async_copy(
                bufs[b], out_hbm.at[pl.ds(base + off, CHUNK)], sws[b]
            )

        def write_wait(b):
            pltpu.make_async_copy(
                bufs[b], out_hbm.at[pl.ds(base, CHUNK)], sws[b]
            ).wait()

        gather_start(0, 0)
        gather_wait(0)
        write_start(0, 0)
        gather_start(1, 1)

        def group(jj, carry):
            for i in range(2):
                k = 1 + 2 * jj + i
                b = (1 + i) % 2
                gather_wait(b)
                write_start(k, b)
                write_wait(1 - b)
                gather_start(k + 1, 1 - b)
            return carry

        lax.fori_loop(0, (NCHUNKS - 2) // 2, group, 0)

        gather_wait(1)
        write_start(NCHUNKS - 1, 1)
        write_wait(0)
        write_wait(1)

    return _gather_sc


def kernel(input):
    x = input
    neg_idxs = _neg_indices()
    perm = jnp.transpose(
        neg_idxs.reshape(BSZ, SEQ, TOTAL_NEG), (2, 0, 1)
    ).reshape(-1)
    flat = x.reshape(NUM_ROWS, DIM)
    out = _build_gather_sc()(flat, perm)
    negs = out.reshape(TOTAL_NEG, BSZ, SEQ, DIM)
    return (x, negs, neg_idxs)
